# Initial kernel scaffold; baseline (speedup 1.0000x reference)
import jax
import jax.numpy as jnp
from jax.experimental import pallas as pl


def kernel(X, edge_index, Ws, W_out, b_out):
    raise NotImplementedError("write your pallas kernel here")


# speedup vs baseline: 4.3935x; 4.3935x over previous
import functools

import jax
import jax.numpy as jnp
from jax import lax
from jax.experimental import pallas as pl
from jax.experimental.pallas import tpu as pltpu
from jax.experimental.pallas import tpu_sc as plsc

N = 10000
E = 320000
F = 128
C = 40
YT = 0.5
KT = 1.0

NCORES = 2
NSUB = 16
NTILES = NCORES * NSUB
CHUNK = 128
NCHUNKS = 79
EPAD = NTILES * NCHUNKS * CHUNK
DUMP_ROW = N
SPM_ROWS = 10240
ZROWS = SPM_ROWS // NSUB

_SC_MESH = plsc.VectorSubcoreMesh(core_axis_name="c", subcore_axis_name="s")


@functools.partial(
    pl.kernel,
    mesh=_SC_MESH,
    out_type=jax.ShapeDtypeStruct((NCORES, SPM_ROWS, F), jnp.float32),
    scratch_types=[
        pltpu.VMEM((NCHUNKS, CHUNK), jnp.int32),
        pltpu.VMEM((NCHUNKS, CHUNK), jnp.int32),
        pltpu.VMEM((CHUNK, F), jnp.float32),
        pltpu.VMEM_SHARED((SPM_ROWS, F), jnp.float32),
        pltpu.SemaphoreType.DMA,
    ],
)
def _sc_spmm(h_hbm, src_hbm, dst_hbm, zeros_hbm, out_hbm,
             src_v, dst_v, rows_v, agg_s, sem):
    cid = lax.axis_index("c")
    sid = lax.axis_index("s")
    wid = cid * NSUB + sid
    pltpu.sync_copy(zeros_hbm, agg_s.at[pl.ds(sid * ZROWS, ZROWS)])
    pltpu.sync_copy(src_hbm.at[wid], src_v)
    pltpu.sync_copy(dst_hbm.at[wid], dst_v)
    plsc.subcore_barrier()

    def body(j, carry):
        pltpu.async_copy(h_hbm.at[src_v.at[j]], rows_v, sem).wait()
        pltpu.sync_copy(rows_v, agg_s.at[dst_v.at[j]], add=True)
        return carry

    lax.fori_loop(0, NCHUNKS, body, 0)
    plsc.subcore_barrier()
    base = sid * ZROWS
    for k in range(ZROWS // CHUNK):
        pltpu.sync_copy(agg_s.at[pl.ds(base + k * CHUNK, CHUNK)], rows_v)
        pltpu.sync_copy(rows_v, out_hbm.at[cid, pl.ds(base + k * CHUNK, CHUNK)])


@functools.partial(
    pl.kernel,
    mesh=_SC_MESH,
    out_type=jax.ShapeDtypeStruct((NCORES, SPM_ROWS, F), jnp.float32),
    scratch_types=[
        pltpu.VMEM((NCHUNKS, CHUNK), jnp.int32),
        pltpu.VMEM((CHUNK, F), jnp.float32),
        pltpu.VMEM_SHARED((SPM_ROWS, F), jnp.float32),
    ],
)
def _sc_deg(dst_hbm, ones_hbm, zeros_hbm, out_hbm,
            dst_v, ones_v, deg_s):
    cid = lax.axis_index("c")
    sid = lax.axis_index("s")
    wid = cid * NSUB + sid
    pltpu.sync_copy(zeros_hbm, deg_s.at[pl.ds(sid * ZROWS, ZROWS)])
    pltpu.sync_copy(ones_hbm, ones_v)
    pltpu.sync_copy(dst_hbm.at[wid], dst_v)
    plsc.subcore_barrier()

    def body(j, carry):
        pltpu.sync_copy(ones_v, deg_s.at[dst_v.at[j]], add=True)
        return carry

    lax.fori_loop(0, NCHUNKS, body, 0)
    plsc.subcore_barrier()
    base = sid * ZROWS
    for k in range(ZROWS // CHUNK):
        pltpu.sync_copy(deg_s.at[pl.ds(base + k * CHUNK, CHUNK)], ones_v)
        pltpu.sync_copy(ones_v,
                        out_hbm.at[cid, pl.ds(base + k * CHUNK, CHUNK)])


NPAD = SPM_ROWS
BLK = 1024


def _layer_body(agg_ref, deg_ref, w_ref, x_ref, h1_ref, o_ref):
    z = agg_ref[0] + agg_ref[1]
    d = deg_ref[0, :, 0:1] + deg_ref[1, :, 0:1]
    scale = YT / jnp.maximum(d, 1.0)
    y = jnp.dot(z, w_ref[...], preferred_element_type=jnp.float32)
    o_ref[...] = jnp.maximum(y * scale + KT * x_ref[...] - h1_ref[...], 0.0)


_tc_layer = pl.pallas_call(
    _layer_body,
    grid=(NPAD // BLK,),
    in_specs=[
        pl.BlockSpec((NCORES, BLK, F), lambda i: (0, i, 0)),
        pl.BlockSpec((NCORES, BLK, F), lambda i: (0, i, 0)),
        pl.BlockSpec((F, F), lambda i: (0, 0)),
        pl.BlockSpec((BLK, F), lambda i: (i, 0)),
        pl.BlockSpec((BLK, F), lambda i: (i, 0)),
    ],
    out_specs=pl.BlockSpec((BLK, F), lambda i: (i, 0)),
    out_shape=jax.ShapeDtypeStruct((NPAD, F), jnp.float32),
)


def _out_body(h_ref, w_ref, b_ref, o_ref):
    logits = jnp.dot(h_ref[...], w_ref[...],
                     preferred_element_type=jnp.float32) + b_ref[...]
    col = lax.broadcasted_iota(jnp.int32, logits.shape, 1)
    valid = col < C
    masked = jnp.where(valid, logits, -jnp.inf)
    m = jnp.max(masked, axis=1, keepdims=True)
    e = jnp.where(valid, jnp.exp(masked - m), 0.0)
    lse = jnp.log(jnp.sum(e, axis=1, keepdims=True)) + m
    o_ref[...] = logits - lse


_tc_out = pl.pallas_call(
    _out_body,
    grid=(NPAD // BLK,),
    in_specs=[
        pl.BlockSpec((BLK, F), lambda i: (i, 0)),
        pl.BlockSpec((F, 128), lambda i: (0, 0)),
        pl.BlockSpec((1, 128), lambda i: (0, 0)),
    ],
    out_specs=pl.BlockSpec((BLK, 128), lambda i: (i, 0)),
    out_shape=jax.ShapeDtypeStruct((NPAD, 128), jnp.float32),
)


def kernel(X, edge_index, Ws, W_out, b_out):
    src = edge_index[0]
    dst = edge_index[1]
    pad = EPAD - E
    src_r = jnp.concatenate(
        [src, jnp.zeros((pad,), jnp.int32)]).reshape(NTILES, NCHUNKS, CHUNK)
    dst_r = jnp.concatenate(
        [dst, jnp.full((pad,), DUMP_ROW, jnp.int32)]).reshape(
            NTILES, NCHUNKS, CHUNK)
    zeros_f = jnp.zeros((ZROWS, F), jnp.float32)
    ones_d = jnp.ones((CHUNK, F), jnp.float32)
    Xp = jnp.concatenate([X, jnp.zeros((NPAD - N, F), jnp.float32)])

    deg_parts = _sc_deg(dst_r, ones_d, zeros_f)

    def prop(H):
        return _sc_spmm(H, src_r, dst_r, zeros_f)

    Ha = _tc_layer(prop(Xp), deg_parts, Ws[0], Xp, Xp)
    Hb = _tc_layer(prop(Ha), deg_parts, Ws[0], Xp, Xp)
    Hc = _tc_layer(prop(Hb), deg_parts, Ws[1], Xp, Xp)
    Hd = _tc_layer(prop(Hc), deg_parts, Ws[2], Xp, Hb)
    He = _tc_layer(prop(Hd), deg_parts, Ws[3], Xp, Hc)

    W_p = jnp.zeros((F, 128), jnp.float32).at[:, :C].set(W_out)
    b_p = jnp.zeros((1, 128), jnp.float32).at[0, :C].set(b_out)
    out = _tc_out(He, W_p, b_p)
    return out[:N, :C]
